# baseline (speedup 1.0000x reference)
import functools

import jax
import jax.numpy as jnp
from jax import lax
from jax.experimental import pallas as pl
from jax.experimental.pallas import tpu as pltpu
from jax.experimental.pallas import tpu_sc as plsc

_N_NODES = 100000
_D = 128
_N_LEAVES = 5000
_LPAD = 5120


def _make_sc_gather():
    info = plsc.get_sparse_core_info()
    nc, ns = info.num_cores, info.num_subcores
    nw = nc * ns
    b_per_w = _LPAD // nw
    mesh = plsc.VectorSubcoreMesh(core_axis_name="c", subcore_axis_name="s")
    @functools.partial(
        pl.kernel,
        mesh=mesh,
        out_type=jax.ShapeDtypeStruct((_N_LEAVES, _D), jnp.float32),
        scratch_types=[
            pltpu.VMEM((b_per_w,), jnp.int32),
            pltpu.VMEM((b_per_w, _D), jnp.float32),
            pltpu.SemaphoreType.DMA,
        ],
    )
    def gather_k(table_hbm, idx_hbm, out_hbm, idx_v, rows_v, sem):
        wid = lax.axis_index("s") * nc + lax.axis_index("c")
        base = jnp.minimum(wid * b_per_w, _N_LEAVES - b_per_w)
        pltpu.sync_copy(idx_hbm.at[pl.ds(base, b_per_w)], idx_v)
        pltpu.async_copy(table_hbm.at[idx_v], rows_v, sem).wait()
        pltpu.sync_copy(rows_v, out_hbm.at[pl.ds(base, b_per_w)])

    return gather_k


_sc_gather_cache = []


def _sc_gather(table, idx):
    if not _sc_gather_cache:
        _sc_gather_cache.append(_make_sc_gather())
    return _sc_gather_cache[0](table, idx)


_SUM_BLOCK = 20000


def _sum_body(x_ref, out_ref):
    @pl.when(pl.program_id(0) == 0)
    def _():
        out_ref[...] = jnp.zeros_like(out_ref)

    out_ref[...] += jnp.sum(x_ref[...], axis=0, keepdims=True)


def _col_sum(node_embs):
    return pl.pallas_call(
        _sum_body,
        grid=(_N_NODES // _SUM_BLOCK,),
        in_specs=[pl.BlockSpec((_SUM_BLOCK, _D), lambda i: (i, 0))],
        out_specs=pl.BlockSpec((1, _D), lambda i: (0, 0)),
        out_shape=jax.ShapeDtypeStruct((1, _D), jnp.float32),
    )(node_embs)


_MLP_BLOCK = 1000
_MLP_STEPS = _N_LEAVES // _MLP_BLOCK


def _mlp_body(g_ref, sum_ref, w1_ref, b1_ref, w2_ref, b2_ref, out_ref, s_ref):
    i = pl.program_id(0)
    graph = sum_ref[...] * (1.0 / _N_NODES)
    w1 = w1_ref[...]
    c = jnp.dot(graph, w1[_D:, :], preferred_element_type=jnp.float32)
    c = c + b1_ref[...]
    h = jnp.dot(g_ref[...], w1[:_D, :], preferred_element_type=jnp.float32)
    h = jnp.maximum(h + c, 0.0)
    s = jnp.sum(h * w2_ref[...], axis=1, keepdims=True) + b2_ref[0, 0]
    s_ref[pl.ds(i * _MLP_BLOCK, _MLP_BLOCK), :] = s

    @pl.when(i == _MLP_STEPS - 1)
    def _softmax():
        sa = s_ref[...]
        e = jnp.exp(sa - jnp.max(sa))
        out_ref[...] = (e / jnp.sum(e)).reshape(_N_LEAVES)


def _mlp(g, col_sum, W1, b1, w2_row, b2):
    return pl.pallas_call(
        _mlp_body,
        grid=(_MLP_STEPS,),
        in_specs=[
            pl.BlockSpec((_MLP_BLOCK, _D), lambda i: (i, 0)),
            pl.BlockSpec((1, _D), lambda i: (0, 0)),
            pl.BlockSpec((2 * _D, _D), lambda i: (0, 0)),
            pl.BlockSpec((1, _D), lambda i: (0, 0)),
            pl.BlockSpec((1, _D), lambda i: (0, 0)),
            pl.BlockSpec((1, 1), lambda i: (0, 0)),
        ],
        out_specs=pl.BlockSpec((_N_LEAVES,), lambda i: (0,)),
        out_shape=jax.ShapeDtypeStruct((_N_LEAVES,), jnp.float32),
        scratch_shapes=[pltpu.VMEM((_N_LEAVES, 1), jnp.float32)],
    )(g, col_sum, W1, b1, w2_row, b2)


def kernel(node_embs, leave_inds, W1, b1, W2, b2):
    g = _sc_gather(node_embs, leave_inds.astype(jnp.int32))
    col_sum = _col_sum(node_embs)
    return _mlp(g, col_sum, W1, b1.reshape(1, _D), W2.reshape(1, _D),
                b2.reshape(1, 1))


# speedup vs baseline: 1.0451x; 1.0451x over previous
import functools

import jax
import jax.numpy as jnp
from jax import lax
from jax.experimental import pallas as pl
from jax.experimental.pallas import tpu as pltpu
from jax.experimental.pallas import tpu_sc as plsc

_N_NODES = 100000
_D = 128
_N_LEAVES = 5000
_LPAD = 5120


def _make_sc_gather():
    info = plsc.get_sparse_core_info()
    nc, ns = info.num_cores, info.num_subcores
    nw = nc * ns
    b_per_w = _LPAD // nw
    mesh = plsc.VectorSubcoreMesh(core_axis_name="c", subcore_axis_name="s")
    @functools.partial(
        pl.kernel,
        mesh=mesh,
        out_type=jax.ShapeDtypeStruct((_N_LEAVES, _D), jnp.float32),
        scratch_types=[
            pltpu.VMEM((b_per_w,), jnp.int32),
            pltpu.VMEM((b_per_w, _D), jnp.float32),
            pltpu.SemaphoreType.DMA,
        ],
    )
    def gather_k(table_hbm, idx_hbm, out_hbm, idx_v, rows_v, sem):
        wid = lax.axis_index("s") * nc + lax.axis_index("c")
        base = jnp.minimum(wid * b_per_w, _N_LEAVES - b_per_w)
        pltpu.sync_copy(idx_hbm.at[pl.ds(base, b_per_w)], idx_v)
        pltpu.async_copy(table_hbm.at[idx_v], rows_v, sem).wait()
        pltpu.sync_copy(rows_v, out_hbm.at[pl.ds(base, b_per_w)])

    return gather_k


_sc_gather_cache = []


def _sc_gather(table, idx):
    if not _sc_gather_cache:
        _sc_gather_cache.append(_make_sc_gather())
    return _sc_gather_cache[0](table, idx)


_SUM_BLOCK = 20000


def _sum_body(x_ref, out_ref):
    @pl.when(pl.program_id(0) == 0)
    def _():
        out_ref[...] = jnp.zeros_like(out_ref)

    out_ref[...] += jnp.sum(x_ref[...], axis=0, keepdims=True)


def _col_sum(node_embs):
    return pl.pallas_call(
        _sum_body,
        grid=(_N_NODES // _SUM_BLOCK,),
        in_specs=[pl.BlockSpec((_SUM_BLOCK, _D), lambda i: (i, 0))],
        out_specs=pl.BlockSpec((1, _D), lambda i: (0, 0)),
        out_shape=jax.ShapeDtypeStruct((1, _D), jnp.float32),
    )(node_embs)


def _mlp_body(g_ref, sum_ref, w1_ref, b1_ref, w2_ref, b2_ref, out_ref):
    graph = sum_ref[...] * (1.0 / _N_NODES)
    w1 = w1_ref[...]
    c = jnp.dot(graph, w1[_D:, :], preferred_element_type=jnp.float32)
    c = c + b1_ref[...]
    h = jnp.dot(g_ref[...], w1[:_D, :], preferred_element_type=jnp.float32)
    h = jnp.maximum(h + c, 0.0)
    s = jnp.sum(h * w2_ref[...], axis=1, keepdims=True) + b2_ref[0, 0]
    e = jnp.exp(s - jnp.max(s))
    out_ref[...] = (e / jnp.sum(e)).reshape(_N_LEAVES)


def _mlp(g, col_sum, W1, b1, w2_row, b2):
    return pl.pallas_call(
        _mlp_body,
        in_specs=[
            pl.BlockSpec((_N_LEAVES, _D), lambda: (0, 0)),
            pl.BlockSpec((1, _D), lambda: (0, 0)),
            pl.BlockSpec((2 * _D, _D), lambda: (0, 0)),
            pl.BlockSpec((1, _D), lambda: (0, 0)),
            pl.BlockSpec((1, _D), lambda: (0, 0)),
            pl.BlockSpec((1, 1), lambda: (0, 0)),
        ],
        out_specs=pl.BlockSpec((_N_LEAVES,), lambda: (0,)),
        out_shape=jax.ShapeDtypeStruct((_N_LEAVES,), jnp.float32),
    )(g, col_sum, W1, b1, w2_row, b2)


def kernel(node_embs, leave_inds, W1, b1, W2, b2):
    g = _sc_gather(node_embs, leave_inds.astype(jnp.int32))
    col_sum = _col_sum(node_embs)
    return _mlp(g, col_sum, W1, b1.reshape(1, _D), W2.reshape(1, _D),
                b2.reshape(1, 1))


# speedup vs baseline: 1.0521x; 1.0067x over previous
import functools

import jax
import jax.numpy as jnp
from jax import lax
from jax.experimental import pallas as pl
from jax.experimental.pallas import tpu as pltpu
from jax.experimental.pallas import tpu_sc as plsc

_N_NODES = 100000
_D = 128
_N_LEAVES = 5000
_LPAD = 5120


def _make_sc_gather():
    info = plsc.get_sparse_core_info()
    nc, ns = info.num_cores, info.num_subcores
    nw = nc * ns
    b_per_w = _LPAD // nw
    mesh = plsc.VectorSubcoreMesh(core_axis_name="c", subcore_axis_name="s")
    @functools.partial(
        pl.kernel,
        mesh=mesh,
        out_type=jax.ShapeDtypeStruct((_N_LEAVES, _D), jnp.float32),
        scratch_types=[
            pltpu.VMEM((b_per_w,), jnp.int32),
            pltpu.VMEM((b_per_w, _D), jnp.float32),
            pltpu.SemaphoreType.DMA,
        ],
    )
    def gather_k(table_hbm, idx_hbm, out_hbm, idx_v, rows_v, sem):
        wid = lax.axis_index("s") * nc + lax.axis_index("c")
        base = jnp.minimum(wid * b_per_w, _N_LEAVES - b_per_w)
        pltpu.sync_copy(idx_hbm.at[pl.ds(base, b_per_w)], idx_v)
        pltpu.async_copy(table_hbm.at[idx_v], rows_v, sem).wait()
        pltpu.sync_copy(rows_v, out_hbm.at[pl.ds(base, b_per_w)])

    return gather_k


_sc_gather_cache = []


def _sc_gather(table, idx):
    if not _sc_gather_cache:
        _sc_gather_cache.append(_make_sc_gather())
    return _sc_gather_cache[0](table, idx)


_SUM_BLOCK = 25000


def _sum_body(x_ref, out_ref):
    @pl.when(pl.program_id(0) == 0)
    def _():
        out_ref[...] = jnp.zeros_like(out_ref)

    out_ref[...] += jnp.sum(x_ref[...], axis=0, keepdims=True)


def _col_sum(node_embs):
    return pl.pallas_call(
        _sum_body,
        grid=(_N_NODES // _SUM_BLOCK,),
        in_specs=[pl.BlockSpec((_SUM_BLOCK, _D), lambda i: (i, 0))],
        out_specs=pl.BlockSpec((1, _D), lambda i: (0, 0)),
        out_shape=jax.ShapeDtypeStruct((1, _D), jnp.float32),
    )(node_embs)


def _mlp_body(g_ref, sum_ref, w1_ref, b1_ref, w2_ref, out_ref):
    graph = sum_ref[...] * (1.0 / _N_NODES)
    w1 = w1_ref[...]
    c = jnp.dot(graph, w1[_D:, :], preferred_element_type=jnp.float32)
    c = c + b1_ref[...]
    h = jnp.dot(g_ref[...], w1[:_D, :], preferred_element_type=jnp.float32)
    h = jnp.maximum(h + c, 0.0)
    s = jnp.sum(h * w2_ref[...], axis=1, keepdims=True)
    s1 = s.reshape(_N_LEAVES)
    e = jnp.exp(s1 - jnp.max(s1))
    out_ref[...] = e / jnp.sum(e)


def _mlp(g, col_sum, W1, b1, w2_row):
    return pl.pallas_call(
        _mlp_body,
        in_specs=[
            pl.BlockSpec((_N_LEAVES, _D), lambda: (0, 0)),
            pl.BlockSpec((1, _D), lambda: (0, 0)),
            pl.BlockSpec((2 * _D, _D), lambda: (0, 0)),
            pl.BlockSpec((1, _D), lambda: (0, 0)),
            pl.BlockSpec((1, _D), lambda: (0, 0)),
        ],
        out_specs=pl.BlockSpec((_N_LEAVES,), lambda: (0,)),
        out_shape=jax.ShapeDtypeStruct((_N_LEAVES,), jnp.float32),
    )(g, col_sum, W1, b1, w2_row)


def kernel(node_embs, leave_inds, W1, b1, W2, b2):
    g = _sc_gather(node_embs, leave_inds.astype(jnp.int32))
    col_sum = _col_sum(node_embs)
    del b2
    return _mlp(g, col_sum, W1, b1.reshape(1, _D), W2.reshape(1, _D))


# speedup vs baseline: 1.0948x; 1.0406x over previous
import functools

import jax
import jax.numpy as jnp
from jax import lax
from jax.experimental import pallas as pl
from jax.experimental.pallas import tpu as pltpu
from jax.experimental.pallas import tpu_sc as plsc

_N_NODES = 100000
_D = 128
_N_LEAVES = 5000
_LPAD = 5120

_SC_ROWS_PER_W = 800
_SC_CHUNK = 200
_SC_ROWS = 32 * _SC_ROWS_PER_W
_TC_ROWS = _N_NODES - _SC_ROWS


def _make_sc_gather():
    info = plsc.get_sparse_core_info()
    nc, ns = info.num_cores, info.num_subcores
    nw = nc * ns
    b_per_w = _LPAD // nw
    mesh = plsc.VectorSubcoreMesh(core_axis_name="c", subcore_axis_name="s")
    n_chunks = _SC_ROWS_PER_W // _SC_CHUNK

    @functools.partial(
        pl.kernel,
        mesh=mesh,
        out_type=(
            jax.ShapeDtypeStruct((_N_LEAVES, _D), jnp.float32),
            jax.ShapeDtypeStruct((nw, _D), jnp.float32),
        ),
        scratch_types=[
            pltpu.VMEM((b_per_w,), jnp.int32),
            pltpu.VMEM((b_per_w, _D), jnp.float32),
            pltpu.VMEM((_SC_CHUNK, _D), jnp.float32),
            pltpu.VMEM((_SC_CHUNK, _D), jnp.float32),
            pltpu.VMEM((_D,), jnp.float32),
            pltpu.SemaphoreType.DMA,
            pltpu.SemaphoreType.DMA,
            pltpu.SemaphoreType.DMA,
        ],
    )
    def gather_k(table_hbm, idx_hbm, out_hbm, psum_hbm,
                 idx_v, rows_v, buf0, buf1, acc_v, sem, s0, s1):
        wid = lax.axis_index("s") * nc + lax.axis_index("c")

        base = jnp.minimum(wid * b_per_w, _N_LEAVES - b_per_w)
        pltpu.sync_copy(idx_hbm.at[pl.ds(base, b_per_w)], idx_v)
        gat = pltpu.async_copy(table_hbm.at[idx_v], rows_v, sem)

        row0 = _TC_ROWS + wid * _SC_ROWS_PER_W
        bufs, sems = (buf0, buf1), (s0, s1)
        copies = [None, None]
        copies[0] = pltpu.async_copy(
            table_hbm.at[pl.ds(row0, _SC_CHUNK)], buf0, s0)
        acc = [jnp.zeros((16,), jnp.float32) for _ in range(8)]
        for j in range(n_chunks):
            if j + 1 < n_chunks:
                copies[(j + 1) % 2] = pltpu.async_copy(
                    table_hbm.at[pl.ds(row0 + (j + 1) * _SC_CHUNK, _SC_CHUNK)],
                    bufs[(j + 1) % 2], sems[(j + 1) % 2])
            copies[j % 2].wait()
            buf = bufs[j % 2]

            def body(r, carry):
                out = []
                for k in range(8):
                    out.append(carry[k] + buf[r, pl.ds(16 * k, 16)])
                return tuple(out)

            acc = list(lax.fori_loop(0, _SC_CHUNK, body, tuple(acc)))
        for k in range(8):
            acc_v[pl.ds(16 * k, 16)] = acc[k]
        pltpu.sync_copy(acc_v, psum_hbm.at[wid])

        gat.wait()
        pltpu.sync_copy(rows_v, out_hbm.at[pl.ds(base, b_per_w)])

    return gather_k


_sc_gather_cache = []


def _sc_gather(table, idx):
    if not _sc_gather_cache:
        _sc_gather_cache.append(_make_sc_gather())
    return _sc_gather_cache[0](table, idx)


_SUM_BLOCK = 18600


def _sum_body(x_ref, out_ref):
    @pl.when(pl.program_id(0) == 0)
    def _():
        out_ref[...] = jnp.zeros_like(out_ref)

    out_ref[...] += jnp.sum(x_ref[...], axis=0, keepdims=True)


def _col_sum(node_embs):
    return pl.pallas_call(
        _sum_body,
        grid=(_TC_ROWS // _SUM_BLOCK,),
        in_specs=[pl.BlockSpec((_SUM_BLOCK, _D), lambda i: (i, 0))],
        out_specs=pl.BlockSpec((1, _D), lambda i: (0, 0)),
        out_shape=jax.ShapeDtypeStruct((1, _D), jnp.float32),
    )(node_embs)


def _mlp_body(g_ref, sum_ref, psum_ref, w1_ref, b1_ref, w2_ref, out_ref):
    total = sum_ref[...] + jnp.sum(psum_ref[...], axis=0, keepdims=True)
    graph = total * (1.0 / _N_NODES)
    w1 = w1_ref[...]
    c = jnp.dot(graph, w1[_D:, :], preferred_element_type=jnp.float32)
    c = c + b1_ref[...]
    h = jnp.dot(g_ref[...], w1[:_D, :], preferred_element_type=jnp.float32)
    h = jnp.maximum(h + c, 0.0)
    s = jnp.sum(h * w2_ref[...], axis=1, keepdims=True)
    s1 = s.reshape(_N_LEAVES)
    e = jnp.exp(s1 - jnp.max(s1))
    out_ref[...] = e / jnp.sum(e)


def _mlp(g, col_sum, psum, W1, b1, w2_row):
    return pl.pallas_call(
        _mlp_body,
        in_specs=[
            pl.BlockSpec((_N_LEAVES, _D), lambda: (0, 0)),
            pl.BlockSpec((1, _D), lambda: (0, 0)),
            pl.BlockSpec((32, _D), lambda: (0, 0)),
            pl.BlockSpec((2 * _D, _D), lambda: (0, 0)),
            pl.BlockSpec((1, _D), lambda: (0, 0)),
            pl.BlockSpec((1, _D), lambda: (0, 0)),
        ],
        out_specs=pl.BlockSpec((_N_LEAVES,), lambda: (0,)),
        out_shape=jax.ShapeDtypeStruct((_N_LEAVES,), jnp.float32),
    )(g, col_sum, psum, W1, b1, w2_row)


def kernel(node_embs, leave_inds, W1, b1, W2, b2):
    g, psum = _sc_gather(node_embs, leave_inds.astype(jnp.int32))
    col_sum = _col_sum(node_embs)
    del b2
    return _mlp(g, col_sum, psum, W1, b1.reshape(1, _D), W2.reshape(1, _D))


# speedup vs baseline: 1.0956x; 1.0007x over previous
import functools

import jax
import jax.numpy as jnp
from jax import lax
from jax.experimental import pallas as pl
from jax.experimental.pallas import tpu as pltpu
from jax.experimental.pallas import tpu_sc as plsc

_N_NODES = 100000
_D = 128
_N_LEAVES = 5000
_LPAD = 5120

_SC_ROWS_PER_W = 800
_SC_CHUNK = 200
_SC_ROWS = 32 * _SC_ROWS_PER_W
_TC_ROWS = _N_NODES - _SC_ROWS


def _make_sc_gather():
    info = plsc.get_sparse_core_info()
    nc, ns = info.num_cores, info.num_subcores
    nw = nc * ns
    b_per_w = _LPAD // nw
    mesh = plsc.VectorSubcoreMesh(core_axis_name="c", subcore_axis_name="s")
    n_chunks = _SC_ROWS_PER_W // _SC_CHUNK

    @functools.partial(
        pl.kernel,
        mesh=mesh,
        out_type=(
            jax.ShapeDtypeStruct((_N_LEAVES, _D), jnp.float32),
            jax.ShapeDtypeStruct((nw, _D), jnp.float32),
        ),
        scratch_types=[
            pltpu.VMEM((b_per_w,), jnp.int32),
            pltpu.VMEM((b_per_w, _D), jnp.float32),
            pltpu.VMEM((_SC_CHUNK, _D), jnp.float32),
            pltpu.VMEM((_SC_CHUNK, _D), jnp.float32),
            pltpu.VMEM((_D,), jnp.float32),
            pltpu.SemaphoreType.DMA,
            pltpu.SemaphoreType.DMA,
            pltpu.SemaphoreType.DMA,
        ],
    )
    def gather_k(table_hbm, idx_hbm, out_hbm, psum_hbm,
                 idx_v, rows_v, buf0, buf1, acc_v, sem, s0, s1):
        wid = lax.axis_index("s") * nc + lax.axis_index("c")

        base = jnp.minimum(wid * b_per_w, _N_LEAVES - b_per_w)
        pltpu.sync_copy(idx_hbm.at[pl.ds(base, b_per_w)], idx_v)
        gat = pltpu.async_copy(table_hbm.at[idx_v], rows_v, sem)

        row0 = _TC_ROWS + wid * _SC_ROWS_PER_W
        bufs, sems = (buf0, buf1), (s0, s1)
        copies = [None, None]
        copies[0] = pltpu.async_copy(
            table_hbm.at[pl.ds(row0, _SC_CHUNK)], buf0, s0)
        acc = [jnp.zeros((16,), jnp.float32) for _ in range(8)]
        for j in range(n_chunks):
            if j + 1 < n_chunks:
                copies[(j + 1) % 2] = pltpu.async_copy(
                    table_hbm.at[pl.ds(row0 + (j + 1) * _SC_CHUNK, _SC_CHUNK)],
                    bufs[(j + 1) % 2], sems[(j + 1) % 2])
            copies[j % 2].wait()
            buf = bufs[j % 2]

            def body(r, carry):
                out = list(carry)
                for u in range(4):
                    for k in range(8):
                        out[k] = out[k] + buf[4 * r + u, pl.ds(16 * k, 16)]
                return tuple(out)

            acc = list(lax.fori_loop(0, _SC_CHUNK // 4, body, tuple(acc)))
        for k in range(8):
            acc_v[pl.ds(16 * k, 16)] = acc[k]
        pltpu.sync_copy(acc_v, psum_hbm.at[wid])

        gat.wait()
        pltpu.sync_copy(rows_v, out_hbm.at[pl.ds(base, b_per_w)])

    return gather_k


_sc_gather_cache = []


def _sc_gather(table, idx):
    if not _sc_gather_cache:
        _sc_gather_cache.append(_make_sc_gather())
    return _sc_gather_cache[0](table, idx)


_SUM_BLOCK = 18600


def _sum_body(x_ref, out_ref):
    @pl.when(pl.program_id(0) == 0)
    def _():
        out_ref[...] = jnp.zeros_like(out_ref)

    out_ref[...] += jnp.sum(x_ref[...], axis=0, keepdims=True)


def _col_sum(node_embs):
    return pl.pallas_call(
        _sum_body,
        grid=(_TC_ROWS // _SUM_BLOCK,),
        in_specs=[pl.BlockSpec((_SUM_BLOCK, _D), lambda i: (i, 0))],
        out_specs=pl.BlockSpec((1, _D), lambda i: (0, 0)),
        out_shape=jax.ShapeDtypeStruct((1, _D), jnp.float32),
    )(node_embs)


def _mlp_body(g_ref, sum_ref, psum_ref, w1_ref, b1_ref, w2_ref, out_ref):
    total = sum_ref[...] + jnp.sum(psum_ref[...], axis=0, keepdims=True)
    graph = total * (1.0 / _N_NODES)
    w1 = w1_ref[...]
    c = jnp.dot(graph, w1[_D:, :], preferred_element_type=jnp.float32)
    c = c + b1_ref[...]
    h = jnp.dot(g_ref[...], w1[:_D, :], preferred_element_type=jnp.float32)
    h = jnp.maximum(h + c, 0.0)
    s = jnp.sum(h * w2_ref[...], axis=1, keepdims=True)
    s1 = s.reshape(_N_LEAVES)
    e = jnp.exp(s1 - jnp.max(s1))
    out_ref[...] = e / jnp.sum(e)


def _mlp(g, col_sum, psum, W1, b1, w2_row):
    return pl.pallas_call(
        _mlp_body,
        in_specs=[
            pl.BlockSpec((_N_LEAVES, _D), lambda: (0, 0)),
            pl.BlockSpec((1, _D), lambda: (0, 0)),
            pl.BlockSpec((32, _D), lambda: (0, 0)),
            pl.BlockSpec((2 * _D, _D), lambda: (0, 0)),
            pl.BlockSpec((1, _D), lambda: (0, 0)),
            pl.BlockSpec((1, _D), lambda: (0, 0)),
        ],
        out_specs=pl.BlockSpec((_N_LEAVES,), lambda: (0,)),
        out_shape=jax.ShapeDtypeStruct((_N_LEAVES,), jnp.float32),
    )(g, col_sum, psum, W1, b1, w2_row)


def kernel(node_embs, leave_inds, W1, b1, W2, b2):
    g, psum = _sc_gather(node_embs, leave_inds.astype(jnp.int32))
    col_sum = _col_sum(node_embs)
    del b2
    return _mlp(g, col_sum, psum, W1, b1.reshape(1, _D), W2.reshape(1, _D))
